# Initial kernel scaffold; baseline (speedup 1.0000x reference)
import jax
import jax.numpy as jnp
from jax.experimental import pallas as pl


def kernel(query, key, value, decoder_segment_ids, key_pages, value_pages, page_map, lengths):
    raise NotImplementedError("write your pallas kernel here")


# speedup vs baseline: 6.7170x; 6.7170x over previous
import functools

import jax
import jax.numpy as jnp
from jax.experimental import pallas as pl
from jax.experimental.pallas import tpu as pltpu

B, NH, NKV, D = 16, 32, 8, 128
T, PPS, NP = 16, 128, 4096
G = NH // NKV
CP = 16
CHTOK = CP * T
NCHUNK = PPS // CP
NEG = -1e10


def _attn_kernel(
    pm_smem,
    len_smem,
    pidx_smem,
    off_smem,
    hits_smem,
    nv_smem,
    q_ref,
    knew_ref,
    vnew_ref,
    kp_hbm,
    vp_hbm,
    o_ref,
    kbuf,
    vbuf,
    sems,
):
    b = pl.program_id(0)
    nv = nv_smem[b]
    length = len_smem[b]

    def issue(c, s):
        for j in range(CP):
            pid = pm_smem[b, c * CP + j]
            pltpu.make_async_copy(kp_hbm.at[:, pid], kbuf.at[s, :, j], sems.at[s, 0]).start()
            pltpu.make_async_copy(vp_hbm.at[:, pid], vbuf.at[s, :, j], sems.at[s, 1]).start()

    def wait(c, s):
        for j in range(CP):
            pid = pm_smem[b, c * CP + j]
            pltpu.make_async_copy(kp_hbm.at[:, pid], kbuf.at[s, :, j], sems.at[s, 0]).wait()
            pltpu.make_async_copy(vp_hbm.at[:, pid], vbuf.at[s, :, j], sems.at[s, 1]).wait()

    issue(0, 0)
    qb = q_ref[0].reshape(NKV, G, D)

    def body(c, carry):
        m, z, acc = carry
        s = jax.lax.rem(c, 2)

        @pl.when(c + 1 < nv)
        def _():
            issue(c + 1, jax.lax.rem(c + 1, 2))

        wait(c, s)

        for j2 in range(B):
            @pl.when(hits_smem[b, c, j2] != 0)
            def _():
                off = off_smem[j2]
                for j in range(CP):
                    @pl.when(pm_smem[b, c * CP + j] == pidx_smem[j2])
                    def _():
                        kbuf[s, :, j, pl.ds(off, 1), :] = knew_ref[:, j2, :].reshape(NKV, 1, D)
                        vbuf[s, :, j, pl.ds(off, 1), :] = vnew_ref[:, j2, :].reshape(NKV, 1, D)

        kc = kbuf[s].reshape(NKV, CHTOK, D)
        vc = vbuf[s].reshape(NKV, CHTOK, D)
        w = jax.lax.dot_general(qb, kc, (((2,), (2,)), ((0,), (0,))),
                                preferred_element_type=jnp.float32)
        pos = c * CHTOK + jax.lax.broadcasted_iota(jnp.int32, (1, 1, CHTOK), 2)
        w = jnp.where(pos < length, w, NEG)
        m_new = jnp.maximum(m, jnp.max(w, axis=2, keepdims=True))
        alpha = jnp.exp(m - m_new)
        e = jnp.exp(w - m_new)
        z = z * alpha + jnp.sum(e, axis=2, keepdims=True)
        pv = jax.lax.dot_general(e, vc, (((2,), (1,)), ((0,), (0,))),
                                 preferred_element_type=jnp.float32)
        acc = acc * alpha + pv
        return m_new, z, acc

    m0 = jnp.full((NKV, G, 1), NEG, jnp.float32)
    z0 = jnp.zeros((NKV, G, 1), jnp.float32)
    a0 = jnp.zeros((NKV, G, D), jnp.float32)
    m, z, acc = jax.lax.fori_loop(0, nv, body, (m0, z0, a0))
    o_ref[0] = (acc / z).reshape(NH, D)


@functools.partial(jax.jit, static_argnames=())
def _paged_attn(query, key, value, key_pages, value_pages, page_map, lengths):
    pos = jnp.maximum(lengths - 1, 0)
    page_idx = page_map[jnp.arange(B), pos // T].astype(jnp.int32)
    offset = (pos % T).astype(jnp.int32)
    knew = jnp.transpose(key[:, 0], (1, 0, 2))
    vnew = jnp.transpose(value[:, 0], (1, 0, 2))
    pm_r = page_map.reshape(B, NCHUNK, CP)
    hits = (pm_r[:, :, :, None] == page_idx[None, None, None, :]).any(2).astype(jnp.int32)
    nvalid = jnp.where(lengths > 0, (lengths + CHTOK - 1) // CHTOK, NCHUNK).astype(jnp.int32)
    q3 = query[:, 0]

    grid_spec = pltpu.PrefetchScalarGridSpec(
        num_scalar_prefetch=6,
        grid=(B,),
        in_specs=[
            pl.BlockSpec((1, NH, D), lambda b, *_: (b, 0, 0)),
            pl.BlockSpec(memory_space=pltpu.MemorySpace.VMEM),
            pl.BlockSpec(memory_space=pltpu.MemorySpace.VMEM),
            pl.BlockSpec(memory_space=pltpu.MemorySpace.HBM),
            pl.BlockSpec(memory_space=pltpu.MemorySpace.HBM),
        ],
        out_specs=pl.BlockSpec((1, NH, D), lambda b, *_: (b, 0, 0)),
        scratch_shapes=[
            pltpu.VMEM((2, NKV, CP, T, D), jnp.float32),
            pltpu.VMEM((2, NKV, CP, T, D), jnp.float32),
            pltpu.SemaphoreType.DMA((2, 2)),
        ],
    )
    out = pl.pallas_call(
        _attn_kernel,
        grid_spec=grid_spec,
        out_shape=jax.ShapeDtypeStruct((B, NH, D), jnp.float32),
    )(page_map, lengths, page_idx, offset, hits, nvalid,
      q3, knew, vnew, key_pages, value_pages)
    return out[:, None, :, :]


def kernel(query, key, value, decoder_segment_ids, key_pages, value_pages, page_map, lengths):
    return _paged_attn(query, key, value, key_pages, value_pages, page_map, lengths)


# speedup vs baseline: 7.3289x; 1.0911x over previous
import functools

import jax
import jax.numpy as jnp
from jax.experimental import pallas as pl
from jax.experimental.pallas import tpu as pltpu

B, NH, NKV, D = 16, 32, 8, 128
T, PPS, NP = 16, 128, 4096
G = NH // NKV
CP = 32
CHTOK = CP * T
NCHUNK = PPS // CP
NEG = -1e10


def _attn_kernel(
    pm_smem,
    len_smem,
    pidx_smem,
    off_smem,
    hits_smem,
    nv_smem,
    q_ref,
    knew_ref,
    vnew_ref,
    kp_hbm,
    vp_hbm,
    o_ref,
    kbuf,
    vbuf,
    sems,
):
    b = pl.program_id(0)
    nv = nv_smem[b]
    length = len_smem[b]

    def issue(c, s):
        for j in range(CP):
            pid = pm_smem[b, c * CP + j]
            pltpu.make_async_copy(kp_hbm.at[:, pid], kbuf.at[s, :, j], sems.at[s, 0]).start()
            pltpu.make_async_copy(vp_hbm.at[:, pid], vbuf.at[s, :, j], sems.at[s, 1]).start()

    def wait(c, s):
        for j in range(CP):
            pid = pm_smem[b, c * CP + j]
            pltpu.make_async_copy(kp_hbm.at[:, pid], kbuf.at[s, :, j], sems.at[s, 0]).wait()
            pltpu.make_async_copy(vp_hbm.at[:, pid], vbuf.at[s, :, j], sems.at[s, 1]).wait()

    issue(0, 0)
    qb = q_ref[0].reshape(NKV, G, D)

    def body(c, carry):
        m, z, acc = carry
        s = jax.lax.rem(c, 2)

        @pl.when(c + 1 < nv)
        def _():
            issue(c + 1, jax.lax.rem(c + 1, 2))

        wait(c, s)

        for j2 in range(B):
            @pl.when(hits_smem[b, c, j2] != 0)
            def _():
                off = off_smem[j2]
                for j in range(CP):
                    @pl.when(pm_smem[b, c * CP + j] == pidx_smem[j2])
                    def _():
                        kbuf[s, :, j, pl.ds(off, 1), :] = knew_ref[:, j2, :].reshape(NKV, 1, D)
                        vbuf[s, :, j, pl.ds(off, 1), :] = vnew_ref[:, j2, :].reshape(NKV, 1, D)

        kc = kbuf[s].reshape(NKV, CHTOK, D)
        vc = vbuf[s].reshape(NKV, CHTOK, D)
        w = jax.lax.dot_general(qb, kc, (((2,), (2,)), ((0,), (0,))),
                                preferred_element_type=jnp.float32)
        pos = c * CHTOK + jax.lax.broadcasted_iota(jnp.int32, (1, 1, CHTOK), 2)
        w = jnp.where(pos < length, w, NEG)
        m_new = jnp.maximum(m, jnp.max(w, axis=2, keepdims=True))
        alpha = jnp.exp(m - m_new)
        e = jnp.exp(w - m_new)
        z = z * alpha + jnp.sum(e, axis=2, keepdims=True)
        pv = jax.lax.dot_general(e, vc, (((2,), (1,)), ((0,), (0,))),
                                 preferred_element_type=jnp.float32)
        acc = acc * alpha + pv
        return m_new, z, acc

    m0 = jnp.full((NKV, G, 1), NEG, jnp.float32)
    z0 = jnp.zeros((NKV, G, 1), jnp.float32)
    a0 = jnp.zeros((NKV, G, D), jnp.float32)
    m, z, acc = jax.lax.fori_loop(0, nv, body, (m0, z0, a0))
    o_ref[0] = (acc / z).reshape(NH, D)


@functools.partial(jax.jit, static_argnames=())
def _paged_attn(query, key, value, key_pages, value_pages, page_map, lengths):
    pos = jnp.maximum(lengths - 1, 0)
    page_idx = page_map[jnp.arange(B), pos // T].astype(jnp.int32)
    offset = (pos % T).astype(jnp.int32)
    knew = jnp.transpose(key[:, 0], (1, 0, 2))
    vnew = jnp.transpose(value[:, 0], (1, 0, 2))
    pm_r = page_map.reshape(B, NCHUNK, CP)
    hits = (pm_r[:, :, :, None] == page_idx[None, None, None, :]).any(2).astype(jnp.int32)
    nvalid = jnp.where(lengths > 0, (lengths + CHTOK - 1) // CHTOK, NCHUNK).astype(jnp.int32)
    q3 = query[:, 0]

    grid_spec = pltpu.PrefetchScalarGridSpec(
        num_scalar_prefetch=6,
        grid=(B,),
        in_specs=[
            pl.BlockSpec((1, NH, D), lambda b, *_: (b, 0, 0)),
            pl.BlockSpec(memory_space=pltpu.MemorySpace.VMEM),
            pl.BlockSpec(memory_space=pltpu.MemorySpace.VMEM),
            pl.BlockSpec(memory_space=pltpu.MemorySpace.HBM),
            pl.BlockSpec(memory_space=pltpu.MemorySpace.HBM),
        ],
        out_specs=pl.BlockSpec((1, NH, D), lambda b, *_: (b, 0, 0)),
        scratch_shapes=[
            pltpu.VMEM((2, NKV, CP, T, D), jnp.float32),
            pltpu.VMEM((2, NKV, CP, T, D), jnp.float32),
            pltpu.SemaphoreType.DMA((2, 2)),
        ],
    )
    out = pl.pallas_call(
        _attn_kernel,
        grid_spec=grid_spec,
        out_shape=jax.ShapeDtypeStruct((B, NH, D), jnp.float32),
    )(page_map, lengths, page_idx, offset, hits, nvalid,
      q3, knew, vnew, key_pages, value_pages)
    return out[:, None, :, :]


def kernel(query, key, value, decoder_segment_ids, key_pages, value_pages, page_map, lengths):
    return _paged_attn(query, key, value, key_pages, value_pages, page_map, lengths)


# speedup vs baseline: 9.4460x; 1.2889x over previous
import functools

import jax
import jax.numpy as jnp
from jax.experimental import pallas as pl
from jax.experimental.pallas import tpu as pltpu

B, NH, NKV, D = 16, 32, 8, 128
T, PPS, NP = 16, 128, 4096
G = NH // NKV
CP = 32
CHTOK = CP * T
NCHUNK = PPS // CP
NEG = -1e10


def _attn_kernel(
    pm_smem,
    len_smem,
    pidx_smem,
    off_smem,
    hits_smem,
    nv_smem,
    base_smem,
    q_ref,
    knew_ref,
    vnew_ref,
    kp_hbm,
    vp_hbm,
    o_ref,
    kbuf,
    vbuf,
    sems,
):
    b = pl.program_id(0)
    nv = nv_smem[b]
    length = len_smem[b]
    base = base_smem[b]

    def issue(bb, c, s):
        for j in range(CP):
            pid = pm_smem[bb, c * CP + j]
            pltpu.make_async_copy(kp_hbm.at[:, pid], kbuf.at[s, :, j], sems.at[s, 0]).start()
            pltpu.make_async_copy(vp_hbm.at[:, pid], vbuf.at[s, :, j], sems.at[s, 1]).start()

    def wait(c, s):
        for j in range(CP):
            pid = pm_smem[b, c * CP + j]
            pltpu.make_async_copy(kp_hbm.at[:, pid], kbuf.at[s, :, j], sems.at[s, 0]).wait()
            pltpu.make_async_copy(vp_hbm.at[:, pid], vbuf.at[s, :, j], sems.at[s, 1]).wait()

    @pl.when(b == 0)
    def _():
        issue(0, 0, 0)

    qb = q_ref[0].reshape(NKV, G, D)

    def body(c, carry):
        m, z, acc = carry
        s = jax.lax.rem(base + c, 2)
        s_next = jax.lax.rem(base + c + 1, 2)

        @pl.when(c + 1 < nv)
        def _():
            issue(b, c + 1, s_next)

        @pl.when((c + 1 == nv) & (b + 1 < B))
        def _():
            issue(b + 1, 0, s_next)

        wait(c, s)

        for j2 in range(B):
            @pl.when(hits_smem[b, c, j2] != 0)
            def _():
                off = off_smem[j2]
                for j in range(CP):
                    @pl.when(pm_smem[b, c * CP + j] == pidx_smem[j2])
                    def _():
                        kbuf[s, :, j, pl.ds(off, 1), :] = knew_ref[:, j2, :].reshape(NKV, 1, D)
                        vbuf[s, :, j, pl.ds(off, 1), :] = vnew_ref[:, j2, :].reshape(NKV, 1, D)

        kc = kbuf[s].reshape(NKV, CHTOK, D)
        vc = vbuf[s].reshape(NKV, CHTOK, D)
        w = jax.lax.dot_general(qb, kc, (((2,), (2,)), ((0,), (0,))),
                                preferred_element_type=jnp.float32)
        pos = c * CHTOK + jax.lax.broadcasted_iota(jnp.int32, (1, 1, CHTOK), 2)
        w = jnp.where(pos < length, w, NEG)
        m_new = jnp.maximum(m, jnp.max(w, axis=2, keepdims=True))
        alpha = jnp.exp(m - m_new)
        e = jnp.exp(w - m_new)
        z = z * alpha + jnp.sum(e, axis=2, keepdims=True)
        pv = jax.lax.dot_general(e, vc, (((2,), (1,)), ((0,), (0,))),
                                 preferred_element_type=jnp.float32)
        acc = acc * alpha + pv
        return m_new, z, acc

    m0 = jnp.full((NKV, G, 1), NEG, jnp.float32)
    z0 = jnp.zeros((NKV, G, 1), jnp.float32)
    a0 = jnp.zeros((NKV, G, D), jnp.float32)
    m, z, acc = jax.lax.fori_loop(0, nv, body, (m0, z0, a0))
    o_ref[0] = (acc / z).reshape(NH, D)


@functools.partial(jax.jit, static_argnames=())
def _paged_attn(query, key, value, key_pages, value_pages, page_map, lengths):
    pos = jnp.maximum(lengths - 1, 0)
    page_idx = page_map[jnp.arange(B), pos // T].astype(jnp.int32)
    offset = (pos % T).astype(jnp.int32)
    knew = jnp.transpose(key[:, 0], (1, 0, 2))
    vnew = jnp.transpose(value[:, 0], (1, 0, 2))
    pm_r = page_map.reshape(B, NCHUNK, CP)
    hits = (pm_r[:, :, :, None] == page_idx[None, None, None, :]).any(2).astype(jnp.int32)
    nvalid = jnp.where(lengths > 0, (lengths + CHTOK - 1) // CHTOK, NCHUNK).astype(jnp.int32)
    base = (jnp.concatenate([jnp.zeros((1,), jnp.int32), jnp.cumsum(nvalid)[:-1]]) % 2).astype(jnp.int32)
    q3 = query[:, 0]

    grid_spec = pltpu.PrefetchScalarGridSpec(
        num_scalar_prefetch=7,
        grid=(B,),
        in_specs=[
            pl.BlockSpec((1, NH, D), lambda b, *_: (b, 0, 0)),
            pl.BlockSpec(memory_space=pltpu.MemorySpace.VMEM),
            pl.BlockSpec(memory_space=pltpu.MemorySpace.VMEM),
            pl.BlockSpec(memory_space=pltpu.MemorySpace.HBM),
            pl.BlockSpec(memory_space=pltpu.MemorySpace.HBM),
        ],
        out_specs=pl.BlockSpec((1, NH, D), lambda b, *_: (b, 0, 0)),
        scratch_shapes=[
            pltpu.VMEM((2, NKV, CP, T, D), jnp.float32),
            pltpu.VMEM((2, NKV, CP, T, D), jnp.float32),
            pltpu.SemaphoreType.DMA((2, 2)),
        ],
    )
    out = pl.pallas_call(
        _attn_kernel,
        grid_spec=grid_spec,
        out_shape=jax.ShapeDtypeStruct((B, NH, D), jnp.float32),
    )(page_map, lengths, page_idx, offset, hits, nvalid, base,
      q3, knew, vnew, key_pages, value_pages)
    return out[:, None, :, :]


def kernel(query, key, value, decoder_segment_ids, key_pages, value_pages, page_map, lengths):
    return _paged_attn(query, key, value, key_pages, value_pages, page_map, lengths)


# speedup vs baseline: 10.6052x; 1.1227x over previous
import functools

import jax
import jax.numpy as jnp
from jax.experimental import pallas as pl
from jax.experimental.pallas import tpu as pltpu

B, NH, NKV, D = 16, 32, 8, 128
T, PPS, NP = 16, 128, 4096
G = NH // NKV
CP = 32
CHTOK = CP * T
NCHUNK = PPS // CP
NSLOT = 3
NEG = -1e10


def _attn_kernel(
    pm_smem,
    len_smem,
    pidx_smem,
    off_smem,
    hits_smem,
    nv_smem,
    gstart_smem,
    gb_smem,
    gc_smem,
    q_ref,
    knew_ref,
    vnew_ref,
    kp_hbm,
    vp_hbm,
    o_ref,
    kbuf,
    vbuf,
    sems,
):
    b = pl.program_id(0)
    nv = nv_smem[b]
    length = len_smem[b]
    gbase = gstart_smem[b]
    gtot = gstart_smem[B]

    def issue(bb, c, s):
        for j in range(CP):
            pid = pm_smem[bb, c * CP + j]
            pltpu.make_async_copy(kp_hbm.at[:, pid], kbuf.at[s, :, j], sems.at[s, 0]).start()
            pltpu.make_async_copy(vp_hbm.at[:, pid], vbuf.at[s, :, j], sems.at[s, 1]).start()

    def issue_global(g):
        @pl.when(g < gtot)
        def _():
            issue(gb_smem[g], gc_smem[g], jax.lax.rem(g, NSLOT))

    def wait(s):
        pltpu.make_async_copy(kp_hbm.at[:, 0:CP], kbuf.at[s], sems.at[s, 0]).wait()
        pltpu.make_async_copy(vp_hbm.at[:, 0:CP], vbuf.at[s], sems.at[s, 1]).wait()

    @pl.when(b == 0)
    def _():
        for g0 in range(NSLOT - 1):
            issue_global(jnp.int32(g0))

    qb = q_ref[0].reshape(NKV, G, D)

    def body(c, carry):
        m, z, acc = carry
        g = gbase + c
        s = jax.lax.rem(g, NSLOT)
        issue_global(g + NSLOT - 1)
        wait(s)

        for j2 in range(B):
            @pl.when(hits_smem[b, c, j2] != 0)
            def _():
                off = off_smem[j2]
                for j in range(CP):
                    @pl.when(pm_smem[b, c * CP + j] == pidx_smem[j2])
                    def _():
                        kbuf[s, :, j, pl.ds(off, 1), :] = knew_ref[:, j2, :].reshape(NKV, 1, D)
                        vbuf[s, :, j, pl.ds(off, 1), :] = vnew_ref[:, j2, :].reshape(NKV, 1, D)

        kc = kbuf[s].reshape(NKV, CHTOK, D)
        vc = vbuf[s].reshape(NKV, CHTOK, D)
        w = jax.lax.dot_general(qb, kc, (((2,), (2,)), ((0,), (0,))),
                                preferred_element_type=jnp.float32)
        pos = c * CHTOK + jax.lax.broadcasted_iota(jnp.int32, (1, 1, CHTOK), 2)
        w = jnp.where(pos < length, w, NEG)
        m_new = jnp.maximum(m, jnp.max(w, axis=2, keepdims=True))
        alpha = jnp.exp(m - m_new)
        e = jnp.exp(w - m_new)
        z = z * alpha + jnp.sum(e, axis=2, keepdims=True)
        pv = jax.lax.dot_general(e, vc, (((2,), (1,)), ((0,), (0,))),
                                 preferred_element_type=jnp.float32)
        acc = acc * alpha + pv
        return m_new, z, acc

    m0 = jnp.full((NKV, G, 1), NEG, jnp.float32)
    z0 = jnp.zeros((NKV, G, 1), jnp.float32)
    a0 = jnp.zeros((NKV, G, D), jnp.float32)
    m, z, acc = jax.lax.fori_loop(0, nv, body, (m0, z0, a0))
    o_ref[0] = (acc / z).reshape(NH, D)


@functools.partial(jax.jit, static_argnames=())
def _paged_attn(query, key, value, key_pages, value_pages, page_map, lengths):
    pos = jnp.maximum(lengths - 1, 0)
    page_idx = page_map[jnp.arange(B), pos // T].astype(jnp.int32)
    offset = (pos % T).astype(jnp.int32)
    knew = jnp.transpose(key[:, 0], (1, 0, 2))
    vnew = jnp.transpose(value[:, 0], (1, 0, 2))
    pm_r = page_map.reshape(B, NCHUNK, CP)
    hits = (pm_r[:, :, :, None] == page_idx[None, None, None, :]).any(2).astype(jnp.int32)
    nvalid = jnp.where(lengths > 0, (lengths + CHTOK - 1) // CHTOK, NCHUNK).astype(jnp.int32)
    gstart = jnp.concatenate([jnp.zeros((1,), jnp.int32),
                              jnp.cumsum(nvalid).astype(jnp.int32)])
    gidx = jnp.arange(B * NCHUNK, dtype=jnp.int32)
    gb = jnp.clip(jnp.searchsorted(gstart[1:], gidx, side='right'), 0, B - 1).astype(jnp.int32)
    gc = (gidx - gstart[gb]).astype(jnp.int32)
    q3 = query[:, 0]

    grid_spec = pltpu.PrefetchScalarGridSpec(
        num_scalar_prefetch=9,
        grid=(B,),
        in_specs=[
            pl.BlockSpec((1, NH, D), lambda b, *_: (b, 0, 0)),
            pl.BlockSpec(memory_space=pltpu.MemorySpace.VMEM),
            pl.BlockSpec(memory_space=pltpu.MemorySpace.VMEM),
            pl.BlockSpec(memory_space=pltpu.MemorySpace.HBM),
            pl.BlockSpec(memory_space=pltpu.MemorySpace.HBM),
        ],
        out_specs=pl.BlockSpec((1, NH, D), lambda b, *_: (b, 0, 0)),
        scratch_shapes=[
            pltpu.VMEM((NSLOT, NKV, CP, T, D), jnp.float32),
            pltpu.VMEM((NSLOT, NKV, CP, T, D), jnp.float32),
            pltpu.SemaphoreType.DMA((NSLOT, 2)),
        ],
    )
    out = pl.pallas_call(
        _attn_kernel,
        grid_spec=grid_spec,
        out_shape=jax.ShapeDtypeStruct((B, NH, D), jnp.float32),
    )(page_map, lengths, page_idx, offset, hits, nvalid, gstart, gb, gc,
      q3, knew, vnew, key_pages, value_pages)
    return out[:, None, :, :]


def kernel(query, key, value, decoder_segment_ids, key_pages, value_pages, page_map, lengths):
    return _paged_attn(query, key, value, key_pages, value_pages, page_map, lengths)
